# baseline (speedup 1.0000x reference)
import functools

import jax
import jax.numpy as jnp
from jax import lax
from jax.experimental import pallas as pl
from jax.experimental.pallas import tpu as pltpu
from jax.experimental.pallas import tpu_sc as plsc

N = 100000
D = 128
G = 512
BA = 4000
NB = N // BA

NC = 2
NS = 16
NW = NC * NS
CH = 3136
BLK = 112
NFULL = CH // BLK
LASTW_FULL = (N - 31 * CH) // BLK
TAIL = N - 31 * CH - LASTW_FULL * BLK


def _pass_a(z_ref, w1_ref, b1_ref, w2_ref, b2_ref, a_ref, m_ref):
    i = pl.program_id(0)
    h = jnp.tanh(
        lax.dot_general(z_ref[...], w1_ref[...], (((1,), (1,)), ((), ())),
                        preferred_element_type=jnp.float32)
        + b1_ref[...][None, :])
    al = lax.dot_general(h, w2_ref[...], (((1,), (1,)), ((), ())),
                         preferred_element_type=jnp.float32)
    a_row = al[:, 0] + b2_ref[0, 0]
    a_ref[0, 0, :] = a_row

    @pl.when(i == 0)
    def _():
        m_ref[0, 0] = -jnp.inf

    m_ref[0, 0] = jnp.maximum(m_ref[0, 0], jnp.max(a_row))


def _scale_rows(zb_ref, eb_ref, erep_ref, m_vec, rows):
    def group(g, _):
        ev = jnp.exp(eb_ref[pl.ds(g * 16, 16)] - m_vec)
        for j in range(16):
            w = jnp.broadcast_to(lax.slice(ev, (j,), (j + 1,)), (16,))
            r = g * 16 + j
            erep_ref[r, :] = w
            for f in range(8):
                zb_ref[r, pl.ds(f * 16, 16)] = zb_ref[r, pl.ds(f * 16, 16)] * w
        return 0

    lax.fori_loop(0, rows // 16, group, 0)


def _sc_body(z_hbm, batch_hbm, a_hbm, m_hbm, outz_hbm, outs_hbm,
             zb, erep, idxb, ab, zt, erept, idxt, at, mv, accz, accs):
    c = lax.axis_index("c")
    s = lax.axis_index("s")
    wid = s * NC + c

    def zrow(r, _):
        for f in range(8):
            zb[r, pl.ds(f * 16, 16)] = jnp.zeros((16,), jnp.float32)
        erep[r, :] = jnp.zeros((16,), jnp.float32)
        return 0

    lax.fori_loop(0, G // NS, zrow, 0)
    pltpu.sync_copy(zb.at[pl.ds(0, G // NS)], accz.at[pl.ds(s * (G // NS), G // NS)])
    pltpu.sync_copy(erep.at[pl.ds(0, G // NS)], accs.at[pl.ds(s * (G // NS), G // NS)])
    pltpu.sync_copy(m_hbm, mv)
    plsc.subcore_barrier()

    m_vec = mv[...]
    nblk = jnp.where(wid < NW - 1, NFULL, LASTW_FULL)

    def block(b, _):
        base = wid * CH + b * BLK
        pltpu.sync_copy(z_hbm.at[pl.ds(base, BLK)], zb)
        pltpu.sync_copy(a_hbm.at[pl.ds(base, BLK)], ab)
        pltpu.sync_copy(batch_hbm.at[pl.ds(base, BLK)], idxb)
        _scale_rows(zb, ab, erep, m_vec, BLK)
        pltpu.sync_copy(zb, accz.at[idxb], add=True)
        pltpu.sync_copy(erep, accs.at[idxb], add=True)
        return 0

    lax.fori_loop(0, nblk, block, 0)

    @pl.when(wid == NW - 1)
    def _():
        base = (NW - 1) * CH + LASTW_FULL * BLK
        pltpu.sync_copy(z_hbm.at[pl.ds(base, TAIL)], zt)
        pltpu.sync_copy(a_hbm.at[pl.ds(base, TAIL)], at)
        pltpu.sync_copy(batch_hbm.at[pl.ds(base, TAIL)], idxt)
        _scale_rows(zt, at, erept, m_vec, TAIL)
        pltpu.sync_copy(zt, accz.at[idxt], add=True)
        pltpu.sync_copy(erept, accs.at[idxt], add=True)

    plsc.subcore_barrier()

    @pl.when(s == 0)
    def _():
        pltpu.sync_copy(accz, outz_hbm.at[c])
        pltpu.sync_copy(accs, outs_hbm.at[c])


def _pass_c(outz_ref, outs_ref, out_ref):
    s1 = outs_ref[0, :, 0:1] + outs_ref[1, :, 0:1] + 1e-8
    out_ref[...] = (outz_ref[0] + outz_ref[1]) / s1


def kernel(z, batch, W1, b1, W2, b2):
    batch = batch.astype(jnp.int32)
    b2_2d = b2.reshape(1, 1)

    a3d, m = pl.pallas_call(
        _pass_a,
        grid=(NB,),
        in_specs=[
            pl.BlockSpec((BA, D), lambda i: (i, 0)),
            pl.BlockSpec((D, D), lambda i: (0, 0)),
            pl.BlockSpec((D,), lambda i: (0,)),
            pl.BlockSpec((1, D), lambda i: (0, 0)),
            pl.BlockSpec((1, 1), lambda i: (0, 0), memory_space=pltpu.SMEM),
        ],
        out_specs=[
            pl.BlockSpec((1, 1, BA), lambda i: (i, 0, 0)),
            pl.BlockSpec((1, 1), lambda i: (0, 0), memory_space=pltpu.SMEM),
        ],
        out_shape=[
            jax.ShapeDtypeStruct((NB, 1, BA), jnp.float32),
            jax.ShapeDtypeStruct((1, 1), jnp.float32),
        ],
    )(z, W1, b1, W2, b2_2d)

    a = a3d.reshape(N)
    m16 = jnp.broadcast_to(m.reshape(1), (16,))

    sc = functools.partial(
        pl.kernel,
        out_type=[
            jax.ShapeDtypeStruct((NC, G, D), jnp.float32),
            jax.ShapeDtypeStruct((NC, G, NS), jnp.float32),
        ],
        mesh=plsc.VectorSubcoreMesh(core_axis_name="c", subcore_axis_name="s"),
        scratch_types=[
            pltpu.VMEM((BLK, D), jnp.float32),
            pltpu.VMEM((BLK, NS), jnp.float32),
            pltpu.VMEM((BLK,), jnp.int32),
            pltpu.VMEM((BLK,), jnp.float32),
            pltpu.VMEM((TAIL, D), jnp.float32),
            pltpu.VMEM((TAIL, NS), jnp.float32),
            pltpu.VMEM((TAIL,), jnp.int32),
            pltpu.VMEM((TAIL,), jnp.float32),
            pltpu.VMEM((16,), jnp.float32),
            pltpu.VMEM_SHARED((G, D), jnp.float32),
            pltpu.VMEM_SHARED((G, NS), jnp.float32),
        ],
    )(_sc_body)
    outz, outs = sc(z, batch, a, m16)

    return pl.pallas_call(
        _pass_c,
        in_specs=[
            pl.BlockSpec((NC, G, D), lambda: (0, 0, 0)),
            pl.BlockSpec((NC, G, NS), lambda: (0, 0, 0)),
        ],
        out_specs=pl.BlockSpec((G, D), lambda: (0, 0)),
        out_shape=jax.ShapeDtypeStruct((G, D), jnp.float32),
    )(outz, outs)


# speedup vs baseline: 1.1987x; 1.1987x over previous
import functools

import jax
import jax.numpy as jnp
from jax import lax
from jax.experimental import pallas as pl
from jax.experimental.pallas import tpu as pltpu
from jax.experimental.pallas import tpu_sc as plsc

N = 100000
D = 128
G = 512
BA = 4000
NB = N // BA

NC = 2
NS = 16
NW = NC * NS
CH = 3136
BLK = 112
NFULL = CH // BLK
LASTW_FULL = (N - 31 * CH) // BLK
TAIL = N - 31 * CH - LASTW_FULL * BLK


def _pass_a(z_ref, w1_ref, b1_ref, w2_ref, b2_ref, a_ref, m_ref):
    i = pl.program_id(0)
    h = jnp.tanh(
        lax.dot_general(z_ref[...], w1_ref[...], (((1,), (1,)), ((), ())),
                        preferred_element_type=jnp.float32)
        + b1_ref[...][None, :])
    al = lax.dot_general(h, w2_ref[...], (((1,), (1,)), ((), ())),
                         preferred_element_type=jnp.float32)
    a_ref[...] = al + b2_ref[0, 0]

    @pl.when(i == 0)
    def _():
        m_ref[0, 0] = -jnp.inf

    m_ref[0, 0] = jnp.maximum(m_ref[0, 0], jnp.max(al))


def _scale_rows(zb_ref, eb_ref, erep_ref, m_vec, rows):
    def group(g, _):
        ev = jnp.exp(eb_ref[pl.ds(g * 16, 16)] - m_vec)
        for j in range(16):
            w = jnp.broadcast_to(lax.slice(ev, (j,), (j + 1,)), (16,))
            r = g * 16 + j
            erep_ref[r, :] = w
            for f in range(8):
                zb_ref[r, pl.ds(f * 16, 16)] = zb_ref[r, pl.ds(f * 16, 16)] * w
        return 0

    lax.fori_loop(0, rows // 16, group, 0)


def _sc_body(z_hbm, batch_hbm, a_hbm, m_hbm, outz_hbm, outs_hbm,
             zb0, zb1, erep0, erep1, idxb0, idxb1, ab0, ab1,
             zt, erept, idxt, at, mv, accz, accs,
             sin0, sin1, sout0, sout1, stail):
    c = lax.axis_index("c")
    s = lax.axis_index("s")
    wid = s * NC + c
    bufs = ((zb0, erep0, idxb0, ab0, sin0, sout0),
            (zb1, erep1, idxb1, ab1, sin1, sout1))

    def start_in(base, buf):
        zb, _, idxb, ab, sin, _ = buf
        pltpu.async_copy(z_hbm.at[pl.ds(base, BLK)], zb, sin)
        pltpu.async_copy(a_hbm.at[pl.ds(base, BLK)], ab, sin)
        pltpu.async_copy(batch_hbm.at[pl.ds(base, BLK)], idxb, sin)

    def wait_in(base, buf):
        zb, _, idxb, ab, sin, _ = buf
        pltpu.make_async_copy(z_hbm.at[pl.ds(base, BLK)], zb, sin).wait()
        pltpu.make_async_copy(a_hbm.at[pl.ds(base, BLK)], ab, sin).wait()
        pltpu.make_async_copy(batch_hbm.at[pl.ds(base, BLK)], idxb, sin).wait()

    def start_out(buf):
        zb, erep, idxb, _, _, sout = buf
        pltpu.sync_copy(zb, accz.at[idxb], add=True)
        pltpu.sync_copy(erep, accs.at[idxb], add=True)

    def wait_out(buf):
        pass

    def zrow(r, _):
        for f in range(8):
            zb0[r, pl.ds(f * 16, 16)] = jnp.zeros((16,), jnp.float32)
        erep0[r, :] = jnp.zeros((16,), jnp.float32)
        return 0

    lax.fori_loop(0, G // NS, zrow, 0)
    pltpu.sync_copy(zb0.at[pl.ds(0, G // NS)],
                    accz.at[pl.ds(s * (G // NS), G // NS)])
    pltpu.sync_copy(erep0.at[pl.ds(0, G // NS)],
                    accs.at[pl.ds(s * (G // NS), G // NS)])
    pltpu.sync_copy(m_hbm, mv)

    nblk = jnp.where(wid < NW - 1, NFULL, LASTW_FULL)
    nsuper = nblk // 2
    base_w = wid * CH

    start_in(base_w, bufs[0])
    start_in(base_w + BLK, bufs[1])
    plsc.subcore_barrier()
    m_vec = mv[...]

    def superstep(t, _):
        base0 = base_w + (2 * t) * BLK
        base1 = base0 + BLK
        wait_in(base0, bufs[0])
        _scale_rows(zb0, ab0, erep0, m_vec, BLK)
        start_out(bufs[0])
        wait_in(base1, bufs[1])
        _scale_rows(zb1, ab1, erep1, m_vec, BLK)
        start_out(bufs[1])

        @pl.when(t < nsuper - 1)
        def _():
            wait_out(bufs[0])
            start_in(base0 + 2 * BLK, bufs[0])
            wait_out(bufs[1])
            start_in(base1 + 2 * BLK, bufs[1])

        return 0

    lax.fori_loop(0, nsuper, superstep, 0)
    wait_out(bufs[0])
    wait_out(bufs[1])

    @pl.when(wid == NW - 1)
    def _():
        base = (NW - 1) * CH + LASTW_FULL * BLK
        pltpu.async_copy(z_hbm.at[pl.ds(base, TAIL)], zt, stail)
        pltpu.async_copy(a_hbm.at[pl.ds(base, TAIL)], at, stail)
        pltpu.async_copy(batch_hbm.at[pl.ds(base, TAIL)], idxt, stail)
        pltpu.make_async_copy(z_hbm.at[pl.ds(base, TAIL)], zt, stail).wait()
        pltpu.make_async_copy(a_hbm.at[pl.ds(base, TAIL)], at, stail).wait()
        pltpu.make_async_copy(batch_hbm.at[pl.ds(base, TAIL)], idxt, stail).wait()
        _scale_rows(zt, at, erept, m_vec, TAIL)
        pltpu.sync_copy(zt, accz.at[idxt], add=True)
        pltpu.sync_copy(erept, accs.at[idxt], add=True)

    plsc.subcore_barrier()

    @pl.when(s == 0)
    def _():
        pltpu.sync_copy(accz, outz_hbm.at[c])
        pltpu.sync_copy(accs, outs_hbm.at[c])


def _pass_c(outz_ref, outs_ref, out_ref):
    s1 = outs_ref[0, :, 0:1] + outs_ref[1, :, 0:1] + 1e-8
    out_ref[...] = (outz_ref[0] + outz_ref[1]) / s1


def kernel(z, batch, W1, b1, W2, b2):
    batch = batch.astype(jnp.int32)
    b2_2d = b2.reshape(1, 1)

    a2d, m = pl.pallas_call(
        _pass_a,
        grid=(NB,),
        in_specs=[
            pl.BlockSpec((BA, D), lambda i: (i, 0)),
            pl.BlockSpec((D, D), lambda i: (0, 0)),
            pl.BlockSpec((D,), lambda i: (0,)),
            pl.BlockSpec((1, D), lambda i: (0, 0)),
            pl.BlockSpec((1, 1), lambda i: (0, 0), memory_space=pltpu.SMEM),
        ],
        out_specs=[
            pl.BlockSpec((BA, 1), lambda i: (i, 0)),
            pl.BlockSpec((1, 1), lambda i: (0, 0), memory_space=pltpu.SMEM),
        ],
        out_shape=[
            jax.ShapeDtypeStruct((N, 1), jnp.float32),
            jax.ShapeDtypeStruct((1, 1), jnp.float32),
        ],
    )(z, W1, b1, W2, b2_2d)

    a = a2d.reshape(N)
    m16 = jnp.broadcast_to(m.reshape(1), (16,))

    sc = functools.partial(
        pl.kernel,
        out_type=[
            jax.ShapeDtypeStruct((NC, G, D), jnp.float32),
            jax.ShapeDtypeStruct((NC, G, NS), jnp.float32),
        ],
        mesh=plsc.VectorSubcoreMesh(core_axis_name="c", subcore_axis_name="s"),
        scratch_types=[
            pltpu.VMEM((BLK, D), jnp.float32),
            pltpu.VMEM((BLK, D), jnp.float32),
            pltpu.VMEM((BLK, NS), jnp.float32),
            pltpu.VMEM((BLK, NS), jnp.float32),
            pltpu.VMEM((BLK,), jnp.int32),
            pltpu.VMEM((BLK,), jnp.int32),
            pltpu.VMEM((BLK,), jnp.float32),
            pltpu.VMEM((BLK,), jnp.float32),
            pltpu.VMEM((TAIL, D), jnp.float32),
            pltpu.VMEM((TAIL, NS), jnp.float32),
            pltpu.VMEM((TAIL,), jnp.int32),
            pltpu.VMEM((TAIL,), jnp.float32),
            pltpu.VMEM((16,), jnp.float32),
            pltpu.VMEM_SHARED((G, D), jnp.float32),
            pltpu.VMEM_SHARED((G, NS), jnp.float32),
            pltpu.SemaphoreType.DMA,
            pltpu.SemaphoreType.DMA,
            pltpu.SemaphoreType.DMA,
            pltpu.SemaphoreType.DMA,
            pltpu.SemaphoreType.DMA,
        ],
    )(_sc_body)
    outz, outs = sc(z, batch, a, m16)

    return pl.pallas_call(
        _pass_c,
        in_specs=[
            pl.BlockSpec((NC, G, D), lambda: (0, 0, 0)),
            pl.BlockSpec((NC, G, NS), lambda: (0, 0, 0)),
        ],
        out_specs=pl.BlockSpec((G, D), lambda: (0, 0)),
        out_shape=jax.ShapeDtypeStruct((G, D), jnp.float32),
    )(outz, outs)
